# Initial kernel scaffold; baseline (speedup 1.0000x reference)
import jax
import jax.numpy as jnp
from jax.experimental import pallas as pl


def kernel(feat0, feat1, ntf0, ntf1, edge_index, fc0_W, fc0_b, fc1_W, fc1_b, ntfc0_W, ntfc0_b, ntfc1_W, ntfc1_b, g0_W, g0_attn_l, g0_attn_r, g0_b, g1_W, g1_attn_l, g1_attn_r, g1_b, lines_W, lines_b):
    raise NotImplementedError("write your pallas kernel here")


# speedup vs baseline: 42.4834x; 42.4834x over previous
import functools

import jax
import jax.numpy as jnp
from jax import lax
from jax.experimental import pallas as pl
from jax.experimental.pallas import tpu as pltpu
from jax.experimental.pallas import tpu_sc as plsc

N0, N1 = 60000, 40000
N = N0 + N1
E = 1600000
D_IN = 128
D_NT = 8
H = 32
HH = 16
NUM_CLASSES = 8
NEG = 0.2

NC, NS, L = 2, 16, 16
CH = 8
E_pad = 49 * 32768
ROWS = E_pad // 128
KA = E_pad // (NC * NS * CH * 128)
KB = E_pad // (NS * CH * 128)
N_pad = 100096
NPT = N_pad // NS
NPT8 = NPT // 8


def _proj_body(x_ref, nt_ref, fcW, fcb, ntW, ntb, gW, al, ar,
               fl_ref, fr_ref, el_ref, er_ref):
    x = x_ref[...]
    h1 = jnp.dot(x, fcW[...], preferred_element_type=jnp.float32) + fcb[...][None, :]
    h2 = jnp.dot(nt_ref[...], ntW[...], preferred_element_type=jnp.float32) + ntb[...][None, :]
    gw = gW[...]
    feat = (jnp.dot(h1, gw[:H], preferred_element_type=jnp.float32)
            + jnp.dot(h2, gw[H:], preferred_element_type=jnp.float32))
    fl_ref[...] = feat[:, :HH]
    fr_ref[...] = feat[:, HH:]
    el_ref[...] = jnp.sum(feat * al[...], axis=1, keepdims=True)
    er_ref[...] = jnp.sum(feat * ar[...], axis=1, keepdims=True)


def _proj_call(x, nt, fcW, fcb, ntW, ntb, gW, al, ar, bn):
    n = x.shape[0]
    grid = (n // bn,)
    full = lambda a: pl.BlockSpec(a.shape, lambda i: (0,) * a.ndim)
    return pl.pallas_call(
        _proj_body,
        grid=grid,
        in_specs=[
            pl.BlockSpec((bn, D_IN), lambda i: (i, 0)),
            pl.BlockSpec((bn, D_NT), lambda i: (i, 0)),
            full(fcW), full(fcb), full(ntW), full(ntb), full(gW), full(al), full(ar),
        ],
        out_specs=[
            pl.BlockSpec((bn, HH), lambda i: (i, 0)),
            pl.BlockSpec((bn, HH), lambda i: (i, 0)),
            pl.BlockSpec((bn, 1), lambda i: (i, 0)),
            pl.BlockSpec((bn, 1), lambda i: (i, 0)),
        ],
        out_shape=[
            jax.ShapeDtypeStruct((n, HH), jnp.float32),
            jax.ShapeDtypeStruct((n, HH), jnp.float32),
            jax.ShapeDtypeStruct((n, 1), jnp.float32),
            jax.ShapeDtypeStruct((n, 1), jnp.float32),
        ],
    )(x, nt, fcW, fcb, ntW, ntb, gW, al, ar)


def _mid_body(a0_ref, a1_ref, d0_ref, d1_ref, g0b, g1W, al, ar,
              fl_ref, fr_ref, el_ref, er_ref):
    inv = 1.0 / (d0_ref[...] + d1_ref[...] + 1e-9)
    x = jnp.concatenate([a0_ref[...], a1_ref[...]], axis=1) * inv + g0b[...][None, :]
    x = jnp.where(x > 0, x, jnp.exp(jnp.minimum(x, 0.0)) - 1.0)
    feat = jnp.dot(x, g1W[...], preferred_element_type=jnp.float32)
    fl_ref[...] = feat[:, :HH]
    fr_ref[...] = feat[:, HH:]
    el_ref[...] = jnp.sum(feat * al[...], axis=1, keepdims=True)
    er_ref[...] = jnp.sum(feat * ar[...], axis=1, keepdims=True)


def _mid_call(a0, a1, d0, d1, g0b, g1W, al, ar, bn):
    n = a0.shape[0]
    full = lambda a: pl.BlockSpec(a.shape, lambda i: (0,) * a.ndim)
    return pl.pallas_call(
        _mid_body,
        grid=(n // bn,),
        in_specs=[
            pl.BlockSpec((bn, HH), lambda i: (i, 0)),
            pl.BlockSpec((bn, HH), lambda i: (i, 0)),
            pl.BlockSpec((bn, 1), lambda i: (i, 0)),
            pl.BlockSpec((bn, 1), lambda i: (i, 0)),
            full(g0b), full(g1W), full(al), full(ar),
        ],
        out_specs=[
            pl.BlockSpec((bn, HH), lambda i: (i, 0)),
            pl.BlockSpec((bn, HH), lambda i: (i, 0)),
            pl.BlockSpec((bn, 1), lambda i: (i, 0)),
            pl.BlockSpec((bn, 1), lambda i: (i, 0)),
        ],
        out_shape=[
            jax.ShapeDtypeStruct((n, HH), jnp.float32),
            jax.ShapeDtypeStruct((n, HH), jnp.float32),
            jax.ShapeDtypeStruct((n, 1), jnp.float32),
            jax.ShapeDtypeStruct((n, 1), jnp.float32),
        ],
    )(a0, a1, d0, d1, g0b, g1W, al, ar)


def _fin_body(a0_ref, a1_ref, d0_ref, d1_ref, g1b, linW, linb, logits_ref, h_ref):
    inv = 1.0 / (d0_ref[...] + d1_ref[...] + 1e-9)
    hcat = jnp.concatenate([a0_ref[...], a1_ref[...]], axis=1) * inv + g1b[...][None, :]
    h_ref[...] = hcat
    logits_ref[...] = (jnp.dot(hcat, linW[...], preferred_element_type=jnp.float32)
                       + linb[...][None, :])


def _fin_call(a0, a1, d0, d1, g1b, linW, linb, bn):
    n = a0.shape[0]
    full = lambda a: pl.BlockSpec(a.shape, lambda i: (0,) * a.ndim)
    return pl.pallas_call(
        _fin_body,
        grid=(n // bn,),
        in_specs=[
            pl.BlockSpec((bn, HH), lambda i: (i, 0)),
            pl.BlockSpec((bn, HH), lambda i: (i, 0)),
            pl.BlockSpec((bn, 1), lambda i: (i, 0)),
            pl.BlockSpec((bn, 1), lambda i: (i, 0)),
            full(g1b), full(linW), full(linb),
        ],
        out_specs=[
            pl.BlockSpec((bn, NUM_CLASSES), lambda i: (i, 0)),
            pl.BlockSpec((bn, H), lambda i: (i, 0)),
        ],
        out_shape=[
            jax.ShapeDtypeStruct((n, NUM_CLASSES), jnp.float32),
            jax.ShapeDtypeStruct((n, H), jnp.float32),
        ],
    )(a0, a1, d0, d1, g1b, linW, linb)


_MESH = plsc.VectorSubcoreMesh(core_axis_name="c", subcore_axis_name="s")
_SC_PARAMS = pltpu.CompilerParams(use_tc_tiling_on_sc=False)


@functools.partial(
    pl.kernel,
    out_type=(
        jax.ShapeDtypeStruct((ROWS, 128), jnp.float32),
        jax.ShapeDtypeStruct((NC * N_pad,), jnp.float32),
    ),
    mesh=_MESH,
    scratch_types=[
        pltpu.VMEM_SHARED((N_pad,), jnp.float32),
        pltpu.VMEM_SHARED((N_pad,), jnp.float32),
        pltpu.VMEM_SHARED((N_pad,), jnp.float32),
        pltpu.VMEM((NPT,), jnp.float32),
        pltpu.VMEM((CH, 128), jnp.int32),
        pltpu.VMEM((CH, 128), jnp.int32),
        pltpu.VMEM((CH, 128), jnp.float32),
        pltpu.VMEM((CH, 128), jnp.float32),
        pltpu.VMEM((CH, 128), jnp.float32),
        pltpu.SemaphoreType.DMA,
        pltpu.SemaphoreType.DMA,
    ],
    compiler_params=_SC_PARAMS,
)
def _phase_a(src_hbm, dst_hbm, el_hbm, er_hbm, ex_hbm, dpart_hbm,
             el_s, er_s, den_s, stage, srcv, dstv, elb, erb, exb,
             sem, gsem):
    c = lax.axis_index("c")
    s = lax.axis_index("s")
    wid = s * NC + c
    myslice = pl.ds(s * NPT, NPT)

    def _zero(i, _):
        stage[pl.ds(i * L, L)] = jnp.zeros((L,), jnp.float32)
        return 0
    lax.fori_loop(0, NPT // L, _zero, 0)
    pltpu.sync_copy(stage, den_s.at[myslice])
    pltpu.sync_copy(el_hbm.at[myslice], stage)
    pltpu.sync_copy(stage, el_s.at[myslice])
    pltpu.sync_copy(er_hbm.at[myslice], stage)
    pltpu.sync_copy(stage, er_s.at[myslice])
    plsc.subcore_barrier()

    def _chunk(k, _):
        r0 = (wid * KA + k) * CH
        pltpu.sync_copy(src_hbm.at[pl.ds(r0, CH)], srcv)
        pltpu.sync_copy(dst_hbm.at[pl.ds(r0, CH)], dstv)
        cps = []
        for j in range(CH):
            cps.append(pltpu.async_copy(el_s.at[srcv.at[j]], elb.at[j], gsem))
            cps.append(pltpu.async_copy(er_s.at[dstv.at[j]], erb.at[j], gsem))
        for cp in cps:
            cp.wait()
        for j in range(CH):
            for i in range(128 // L):
                sl = pl.ds(i * L, L)
                x = elb[j, sl] + erb[j, sl]
                x = jnp.maximum(x, NEG * x)
                exb[j, sl] = jnp.exp(x)
        pltpu.sync_copy(exb, ex_hbm.at[pl.ds(r0, CH)])
        for j in range(CH):
            pltpu.sync_copy(exb.at[j], den_s.at[dstv.at[j]], add=True)
        return 0

    lax.fori_loop(0, KA, _chunk, 0)
    plsc.subcore_barrier()
    pltpu.sync_copy(den_s.at[myslice], stage)
    pltpu.sync_copy(stage, dpart_hbm.at[pl.ds(c * N_pad + s * NPT, NPT)])


NZC = 368
NZK = NPT // NZC


@functools.partial(
    pl.kernel,
    out_type=jax.ShapeDtypeStruct((NC, N_pad, HH), jnp.float32),
    mesh=_MESH,
    scratch_types=[
        pltpu.VMEM_SHARED((N_pad, HH), jnp.float32),
        pltpu.VMEM((NZC, HH), jnp.float32),
        pltpu.VMEM((CH, 128), jnp.int32),
        pltpu.VMEM((CH, 128), jnp.int32),
        pltpu.VMEM((CH, 128), jnp.float32),
        pltpu.VMEM((CH, 128), jnp.int32),
        pltpu.VMEM((CH * 128, HH), jnp.float32),
        pltpu.SemaphoreType.DMA,
        pltpu.SemaphoreType.DMA,
    ],
    compiler_params=_SC_PARAMS,
)
def _phase_b(src_hbm, dst_hbm, ex_hbm, feat_hbm, acc_hbm,
             acc_s, zb, srcv, dstv, exv, idxb, rows, sem, gsem):
    c = lax.axis_index("c")
    s = lax.axis_index("s")
    coff = c * N_pad

    def _zero(i, _):
        zb[i, :] = jnp.zeros((HH,), jnp.float32)
        return 0
    lax.fori_loop(0, NZC, _zero, 0)
    for t in range(NZK):
        pltpu.sync_copy(zb, acc_s.at[pl.ds(s * NPT + t * NZC, NZC), :])
    plsc.subcore_barrier()

    def _chunk(k, _):
        r0 = (s * KB + k) * CH
        pltpu.sync_copy(src_hbm.at[pl.ds(r0, CH)], srcv)
        pltpu.sync_copy(dst_hbm.at[pl.ds(r0, CH)], dstv)
        pltpu.sync_copy(ex_hbm.at[pl.ds(r0, CH)], exv)
        for j in range(CH):
            for i in range(128 // L):
                sl = pl.ds(i * L, L)
                idxb[j, sl] = srcv[j, sl] + coff
        cps = [pltpu.async_copy(feat_hbm.at[idxb.at[j]],
                                rows.at[pl.ds(j * 128, 128), :], gsem)
               for j in range(CH)]
        for cp in cps:
            cp.wait()
        for j in range(CH):
            def _scale(g, _):
                w16 = exv[j, pl.ds(g * L, L)]
                base = j * 128 + g * L
                for q in range(L):
                    rows[base + q, :] = rows[base + q, :] * w16[q]
                return 0
            lax.fori_loop(0, 128 // L, _scale, 0)
        for j in range(CH):
            pltpu.sync_copy(rows.at[pl.ds(j * 128, 128), :],
                            acc_s.at[dstv.at[j]], add=True)
        return 0

    lax.fori_loop(0, KB, _chunk, 0)
    plsc.subcore_barrier()
    for t in range(NZK):
        rsl = pl.ds(s * NPT + t * NZC, NZC)
        pltpu.sync_copy(acc_s.at[rsl, :], zb)
        pltpu.sync_copy(zb, acc_hbm.at[c, rsl, :])


def _pad_nodes_1d(x):
    return jnp.pad(x.reshape(-1), (0, N_pad - N))


def _gat_layer_sc(src2, dst2, fl, fr, el, er):
    ex, dparts = _phase_a(src2, dst2, el, er)
    feat_cat = jnp.concatenate([fl, fr], axis=0)
    acc = _phase_b(src2, dst2, ex, feat_cat)
    d0 = dparts[:N_pad][:N].reshape(N, 1)
    d1 = dparts[N_pad:][:N].reshape(N, 1)
    return acc[0, :N, :], acc[1, :N, :], d0, d1


def kernel(feat0, feat1, ntf0, ntf1, edge_index, fc0_W, fc0_b, fc1_W, fc1_b,
           ntfc0_W, ntfc0_b, ntfc1_W, ntfc1_b,
           g0_W, g0_attn_l, g0_attn_r, g0_b,
           g1_W, g1_attn_l, g1_attn_r, g1_b,
           lines_W, lines_b):
    src = edge_index[0].astype(jnp.int32)
    dst = edge_index[1].astype(jnp.int32)
    src2 = jnp.pad(src, (0, E_pad - E)).reshape(ROWS, 128)
    dst2 = jnp.pad(dst, (0, E_pad - E), constant_values=N).reshape(ROWS, 128)

    fl0a, fr0a, el0a, er0a = _proj_call(
        feat0, ntf0, fc0_W, fc0_b, ntfc0_W, ntfc0_b, g0_W, g0_attn_l,
        g0_attn_r, 2000)
    fl0b, fr0b, el0b, er0b = _proj_call(
        feat1, ntf1, fc1_W, fc1_b, ntfc1_W, ntfc1_b, g0_W, g0_attn_l,
        g0_attn_r, 2000)
    fl = jnp.pad(jnp.concatenate([fl0a, fl0b], axis=0), ((0, N_pad - N), (0, 0)))
    fr = jnp.pad(jnp.concatenate([fr0a, fr0b], axis=0), ((0, N_pad - N), (0, 0)))
    el = _pad_nodes_1d(jnp.concatenate([el0a, el0b], axis=0))
    er = _pad_nodes_1d(jnp.concatenate([er0a, er0b], axis=0))

    accL0, accR0, d00, d01 = _gat_layer_sc(src2, dst2, fl, fr, el, er)

    fl1, fr1, el1, er1 = _mid_call(accL0, accR0, d00, d01, g0_b, g1_W,
                                   g1_attn_l, g1_attn_r, 2000)
    fl1 = jnp.pad(fl1, ((0, N_pad - N), (0, 0)))
    fr1 = jnp.pad(fr1, ((0, N_pad - N), (0, 0)))
    el1 = _pad_nodes_1d(el1)
    er1 = _pad_nodes_1d(er1)

    accL1, accR1, d10, d11 = _gat_layer_sc(src2, dst2, fl1, fr1, el1, er1)

    logits, h = _fin_call(accL1, accR1, d10, d11, g1_b, lines_W, lines_b, 2000)
    return (logits, h)
